# baseline (speedup 1.0000x reference)
import jax
import jax.numpy as jnp
from jax.experimental import pallas as pl
from jax.experimental.pallas import tpu as pltpu

DEPTH = 7
B_BLK = 1


def _body(mask_ref, img_ref, oh_ref, img_out):
    for bb in range(B_BLK):
        m = mask_ref[bb]
        for c in range(DEPTH):
            oh_ref[bb, c] = (m == c).astype(jnp.float32)
    img_out[...] = img_ref[...]


@jax.jit
def _fused(masks, img_t):
    b, h, w = masks.shape
    masks = pltpu.with_memory_space_constraint(masks, pltpu.MemorySpace.HBM)
    img_t = pltpu.with_memory_space_constraint(img_t, pltpu.MemorySpace.HBM)
    return pl.pallas_call(
        _body,
        grid=(b // B_BLK,),
        in_specs=[
            pl.BlockSpec((B_BLK, h, w), lambda i: (i, 0, 0)),
            pl.BlockSpec((B_BLK, 3, h, w), lambda i: (i, 0, 0, 0)),
        ],
        out_specs=[
            pl.BlockSpec((B_BLK, DEPTH, h, w), lambda i: (i, 0, 0, 0)),
            pl.BlockSpec((B_BLK, 3, h, w), lambda i: (i, 0, 0, 0)),
        ],
        out_shape=[
            jax.ShapeDtypeStruct((b, DEPTH, h, w), jnp.float32),
            jax.ShapeDtypeStruct(img_t.shape, img_t.dtype),
        ],
        compiler_params=pltpu.CompilerParams(
            dimension_semantics=("parallel",),
            vmem_limit_bytes=60 * 1024 * 1024,
        ),
    )(masks, img_t)


def kernel(images, masks, weights):
    img_t = jnp.transpose(images, (0, 3, 1, 2))
    oh_planes, img_out = _fused(masks, img_t)
    return (
        jnp.transpose(img_out, (0, 2, 3, 1)),
        jnp.transpose(oh_planes, (0, 2, 3, 1)),
        weights,
    )


# speedup vs baseline: 1.0312x; 1.0312x over previous
import jax
import jax.numpy as jnp
from jax.experimental import pallas as pl
from jax.experimental.pallas import tpu as pltpu

DEPTH = 7
B_BLK = 2


def _body(mask_ref, img_ref, oh_ref, img_out):
    for bb in range(B_BLK):
        m = mask_ref[bb]
        for c in range(DEPTH):
            oh_ref[bb, c] = (m == c).astype(jnp.float32)
    img_out[...] = img_ref[...]


@jax.jit
def _fused(masks, img_t):
    b, h, w = masks.shape
    masks = pltpu.with_memory_space_constraint(masks, pltpu.MemorySpace.HBM)
    img_t = pltpu.with_memory_space_constraint(img_t, pltpu.MemorySpace.HBM)
    return pl.pallas_call(
        _body,
        grid=(b // B_BLK,),
        in_specs=[
            pl.BlockSpec((B_BLK, h, w), lambda i: (i, 0, 0)),
            pl.BlockSpec((B_BLK, 3, h, w), lambda i: (i, 0, 0, 0)),
        ],
        out_specs=[
            pl.BlockSpec((B_BLK, DEPTH, h, w), lambda i: (i, 0, 0, 0)),
            pl.BlockSpec((B_BLK, 3, h, w), lambda i: (i, 0, 0, 0)),
        ],
        out_shape=[
            jax.ShapeDtypeStruct((b, DEPTH, h, w), jnp.float32),
            jax.ShapeDtypeStruct(img_t.shape, img_t.dtype),
        ],
        compiler_params=pltpu.CompilerParams(
            dimension_semantics=("parallel",),
            vmem_limit_bytes=60 * 1024 * 1024,
        ),
    )(masks, img_t)


def kernel(images, masks, weights):
    img_t = jnp.transpose(images, (0, 3, 1, 2))
    oh_planes, img_out = _fused(masks, img_t)
    return (
        jnp.transpose(img_out, (0, 2, 3, 1)),
        jnp.transpose(oh_planes, (0, 2, 3, 1)),
        weights,
    )
